# baseline (speedup 1.0000x reference)
import functools

import jax
import jax.numpy as jnp
from jax import lax
from jax.experimental import pallas as pl
from jax.experimental.pallas import tpu as pltpu
from jax.experimental.pallas import tpu_sc as plsc

_LANES = 16
_DIM = 32

_mesh = plsc.VectorSubcoreMesh(core_axis_name="c", subcore_axis_name="s")


@functools.partial(
    pl.kernel,
    mesh=_mesh,
    out_type=jax.ShapeDtypeStruct((_LANES,), jnp.float32),
    scratch_types=[
        pltpu.VMEM((_LANES,), jnp.int32),
        pltpu.VMEM((_DIM,), jnp.float32),
        pltpu.VMEM((_DIM,), jnp.float32),
        pltpu.VMEM((_LANES,), jnp.float32),
        pltpu.SemaphoreType.DMA,
    ],
)
def _sc_dot(idx_hbm, ptab_hbm, utab_hbm, out_hbm, idx_v, prow_v, urow_v, out_v, sem):
    wid = lax.axis_index("s") * 2 + lax.axis_index("c")

    @pl.when(wid == 0)
    def _():
        pltpu.sync_copy(idx_hbm, idx_v)
        iv = idx_v[...]
        poff = iv[0] * _DIM
        uoff = iv[1] * _DIM
        cp_p = pltpu.async_copy(ptab_hbm.at[pl.ds(poff, _DIM)], prow_v, sem)
        cp_u = pltpu.async_copy(utab_hbm.at[pl.ds(uoff, _DIM)], urow_v, sem)
        cp_p.wait()
        cp_u.wait()
        a0 = prow_v[pl.ds(0, _LANES)]
        a1 = prow_v[pl.ds(_LANES, _LANES)]
        b0 = urow_v[pl.ds(0, _LANES)]
        b1 = urow_v[pl.ds(_LANES, _LANES)]
        acc = a0 * b0 + a1 * b1
        s = acc[0]
        for i in range(1, _LANES):
            s = s + acc[i]
        out_v[...] = jnp.full((_LANES,), s, jnp.float32)
        pltpu.sync_copy(out_v, out_hbm)


def kernel(product, user, product_embedding, user_embedding):
    p = jnp.asarray(product, jnp.int32)
    u = jnp.asarray(user, jnp.int32)
    idx = jnp.zeros((_LANES,), jnp.int32).at[0].set(p).at[1].set(u)
    out = _sc_dot(
        idx,
        product_embedding.reshape(-1),
        user_embedding.reshape(-1),
    )
    return out[0]


# speedup vs baseline: 41.4814x; 41.4814x over previous
import functools

import jax
import jax.numpy as jnp
from jax import lax
from jax.experimental import pallas as pl
from jax.experimental.pallas import tpu as pltpu
from jax.experimental.pallas import tpu_sc as plsc

_LANES = 16
_DIM = 32

_mesh = plsc.VectorSubcoreMesh(core_axis_name="c", subcore_axis_name="s")


@functools.partial(
    pl.kernel,
    mesh=_mesh,
    compiler_params=pltpu.CompilerParams(needs_layout_passes=False),
    out_type=jax.ShapeDtypeStruct((_LANES,), jnp.float32),
    scratch_types=[
        pltpu.VMEM((_LANES,), jnp.int32),
        pltpu.VMEM((_DIM, 128), jnp.float32),
        pltpu.VMEM((_DIM, 128), jnp.float32),
        pltpu.VMEM((_LANES,), jnp.float32),
        pltpu.SemaphoreType.DMA,
    ],
)
def _sc_dot(idx_hbm, ptabT_hbm, utabT_hbm, out_hbm, idx_v, pcol_v, ucol_v, out_v, sem):
    wid = lax.axis_index("s") * 2 + lax.axis_index("c")

    @pl.when(wid == 0)
    def _():
        pltpu.sync_copy(idx_hbm, idx_v)
        iv = idx_v[...]
        pidx = iv[0]
        uidx = iv[1]
        pbase = pl.multiple_of((pidx // 128) * 128, 128)
        ubase = pl.multiple_of((uidx // 128) * 128, 128)
        cp_p = pltpu.async_copy(ptabT_hbm.at[:, pl.ds(pbase, 128)], pcol_v, sem)
        cp_u = pltpu.async_copy(utabT_hbm.at[:, pl.ds(ubase, 128)], ucol_v, sem)
        cp_p.wait()
        cp_u.wait()
        lane = lax.iota(jnp.int32, _LANES)
        pc = jnp.full((_LANES,), pidx % 128, jnp.int32)
        uc = jnp.full((_LANES,), uidx % 128, jnp.int32)
        a0 = plsc.load_gather(pcol_v, [lane, pc])
        a1 = plsc.load_gather(pcol_v, [lane + _LANES, pc])
        b0 = plsc.load_gather(ucol_v, [lane, uc])
        b1 = plsc.load_gather(ucol_v, [lane + _LANES, uc])
        acc = a0 * b0 + a1 * b1
        s = acc[0]
        for i in range(1, _LANES):
            s = s + acc[i]
        out_v[...] = jnp.full((_LANES,), s, jnp.float32)
        pltpu.sync_copy(out_v, out_hbm)


def kernel(product, user, product_embedding, user_embedding):
    p = jnp.asarray(product, jnp.int32)
    u = jnp.asarray(user, jnp.int32)
    idx = jnp.zeros((_LANES,), jnp.int32).at[0].set(p).at[1].set(u)
    out = _sc_dot(idx, product_embedding.T, user_embedding.T)
    return out[0]


# speedup vs baseline: 45.1783x; 1.0891x over previous
import functools

import jax
import jax.numpy as jnp
from jax import lax
from jax.experimental import pallas as pl
from jax.experimental.pallas import tpu as pltpu
from jax.experimental.pallas import tpu_sc as plsc

_LANES = 16
_DIM = 32

_mesh = plsc.VectorSubcoreMesh(
    core_axis_name="c", subcore_axis_name="s", num_cores=1, num_subcores=1
)


@functools.partial(
    pl.kernel,
    mesh=_mesh,
    compiler_params=pltpu.CompilerParams(needs_layout_passes=False),
    out_type=jax.ShapeDtypeStruct((_LANES,), jnp.float32),
    scratch_types=[
        pltpu.VMEM((_LANES,), jnp.int32),
        pltpu.VMEM((_DIM, 128), jnp.float32),
        pltpu.VMEM((_DIM, 128), jnp.float32),
        pltpu.VMEM((_LANES,), jnp.float32),
        pltpu.SemaphoreType.DMA,
    ],
)
def _sc_dot(idx_hbm, ptabT_hbm, utabT_hbm, out_hbm, idx_v, pcol_v, ucol_v, out_v, sem):
    if True:
        pltpu.sync_copy(idx_hbm, idx_v)
        iv = idx_v[...]
        pidx = iv[0]
        uidx = iv[1]
        pbase = pl.multiple_of((pidx // 128) * 128, 128)
        ubase = pl.multiple_of((uidx // 128) * 128, 128)
        cp_p = pltpu.async_copy(ptabT_hbm.at[:, pl.ds(pbase, 128)], pcol_v, sem)
        cp_u = pltpu.async_copy(utabT_hbm.at[:, pl.ds(ubase, 128)], ucol_v, sem)
        cp_p.wait()
        cp_u.wait()
        lane = lax.iota(jnp.int32, _LANES)
        pc = jnp.full((_LANES,), pidx % 128, jnp.int32)
        uc = jnp.full((_LANES,), uidx % 128, jnp.int32)
        a0 = plsc.load_gather(pcol_v, [lane, pc])
        a1 = plsc.load_gather(pcol_v, [lane + _LANES, pc])
        b0 = plsc.load_gather(ucol_v, [lane, uc])
        b1 = plsc.load_gather(ucol_v, [lane + _LANES, uc])
        acc = a0 * b0 + a1 * b1
        s = acc[0]
        for i in range(1, _LANES):
            s = s + acc[i]
        out_v[...] = jnp.full((_LANES,), s, jnp.float32)
        pltpu.sync_copy(out_v, out_hbm)


def kernel(product, user, product_embedding, user_embedding):
    p = jnp.asarray(product, jnp.int32)
    u = jnp.asarray(user, jnp.int32)
    idx = jnp.zeros((_LANES,), jnp.int32).at[0].set(p).at[1].set(u)
    out = _sc_dot(idx, product_embedding.T, user_embedding.T)
    return out[0]


# speedup vs baseline: 45.9382x; 1.0168x over previous
import functools

import jax
import jax.numpy as jnp
from jax import lax
from jax.experimental import pallas as pl
from jax.experimental.pallas import tpu as pltpu
from jax.experimental.pallas import tpu_sc as plsc

_LANES = 16
_DIM = 32

_mesh = plsc.VectorSubcoreMesh(
    core_axis_name="c", subcore_axis_name="s", num_cores=1, num_subcores=1
)


@functools.partial(
    pl.kernel,
    mesh=_mesh,
    compiler_params=pltpu.CompilerParams(needs_layout_passes=False),
    out_type=jax.ShapeDtypeStruct((_LANES,), jnp.float32),
    scratch_types=[
        pltpu.VMEM((_LANES,), jnp.int32),
        pltpu.VMEM((_DIM, 128), jnp.float32),
        pltpu.VMEM((_DIM, 128), jnp.float32),
        pltpu.VMEM((_LANES,), jnp.float32),
        pltpu.SemaphoreType.DMA,
    ],
)
def _sc_dot(idx_hbm, ptabT_hbm, utabT_hbm, out_hbm, idx_v, pcol_v, ucol_v, out_v, sem):
    if True:
        pltpu.sync_copy(idx_hbm, idx_v)
        iv = idx_v[...]
        pidx = iv[0]
        uidx = iv[1]
        pbase = pl.multiple_of((pidx // 128) * 128, 128)
        ubase = pl.multiple_of((uidx // 128) * 128, 128)
        cp_p = pltpu.async_copy(ptabT_hbm.at[:, pl.ds(pbase, 128)], pcol_v, sem)
        cp_u = pltpu.async_copy(utabT_hbm.at[:, pl.ds(ubase, 128)], ucol_v, sem)
        cp_p.wait()
        cp_u.wait()
        lane = lax.iota(jnp.int32, _LANES)
        pc = jnp.full((_LANES,), pidx % 128, jnp.int32)
        uc = jnp.full((_LANES,), uidx % 128, jnp.int32)
        a0 = plsc.load_gather(pcol_v, [lane, pc])
        a1 = plsc.load_gather(pcol_v, [lane + _LANES, pc])
        b0 = plsc.load_gather(ucol_v, [lane, uc])
        b1 = plsc.load_gather(ucol_v, [lane + _LANES, uc])
        acc = a0 * b0 + a1 * b1
        s = jnp.sum(acc)
        out_v[...] = jnp.full((_LANES,), s, jnp.float32)
        pltpu.sync_copy(out_v, out_hbm)


def kernel(product, user, product_embedding, user_embedding):
    p = jnp.asarray(product, jnp.int32)
    u = jnp.asarray(user, jnp.int32)
    idx = jnp.zeros((_LANES,), jnp.int32).at[0].set(p).at[1].set(u)
    out = _sc_dot(idx, product_embedding.T, user_embedding.T)
    return out[0]
